# baseline (speedup 1.0000x reference)
import jax
import jax.numpy as jnp
from jax.experimental import pallas as pl
from jax.experimental.pallas import tpu as pltpu

C = 256
_LEVELS = [(48, 48), (24, 24), (12, 12), (6, 6), (3, 3)]
_PWS = [w + 2 for _, w in _LEVELS]
_CH = 1024

_STARTS = []
_cur = 5
for (_h, _w), _pw in zip(_LEVELS, _PWS):
    _s = _cur + (-( _cur + _pw + 1)) % 8
    _STARTS.append(_s)
    _cur = _s + (_h + 2) * _pw
_ROWS = _cur + (-_cur) % 8 + 8

_SPANS = []
_edges = []
for _l, ((_h, _w), _pw, _s) in enumerate(zip(_LEVELS, _PWS, _STARTS)):
    _a = _s + _pw + 1
    _L = (_h - 1) * _pw + _w
    _L += (-_L) % 8
    _edges.append((_a, _a + _L))
    for _c0 in range(_a, _a + _L, _CH):
        _SPANS.append((_l, _c0, min(_c0 + _CH, _a + _L)))
_HOLES = []
_prev = 0
for _a, _b in _edges:
    if _a > _prev:
        _HOLES.append((_prev, _a))
    _prev = _b
_HOLES.append((_prev, _ROWS))


def _mask_chunk(lvl, c0, c1):
    (h, w), pw, s = _LEVELS[lvl], _PWS[lvl], _STARTS[lvl]
    r = jax.lax.broadcasted_iota(jnp.int32, (c1 - c0, 1), 0) + c0
    ww = (r - s) % pw
    return (ww >= 1) & (ww <= w) & (r >= s + pw) & (r < s + (h + 1) * pw)


def _head_kern(xin, tww, towb, headw, headb, out, sa, sbuf):
    for buf in (sa, sbuf):
        for a, b in _HOLES:
            buf[a:b, :] = jnp.zeros((b - a, C), jnp.bfloat16)

    def conv(src, w_ref, base, b_row, dst, relu):
        for lvl, c0, c1 in _SPANS:
            pw = _PWS[lvl]
            acc = None
            for t, (dy, dx) in enumerate(
                    (dy, dx) for dy in (-1, 0, 1) for dx in (-1, 0, 1)):
                off = dy * pw + dx
                d = jnp.dot(src[c0 + off:c1 + off, :], w_ref[0, base + t],
                            preferred_element_type=jnp.float32)
                acc = d if acc is None else acc + d
            y = acc + b_row
            if relu:
                y = jnp.where(_mask_chunk(lvl, c0, c1),
                              jnp.maximum(y, 0.0), 0.0)
                dst[c0:c1, :] = y.astype(jnp.bfloat16)
            else:
                dst[0, c0:c1, :] = y

    src = xin
    for i in range(4):
        dst = sa if i % 2 == 0 else sbuf
        conv(src, tww, 9 * i, towb[0, i:i + 1, :], dst, True)
        src = dst
    conv(src, headw, 0, headb[0, 0:1, :], out, False)


def kernel(feat_p3, feat_p4, feat_p5, feat_p6, feat_p7,
           cls_w0, cls_b0, box_w0, box_b0,
           cls_w1, cls_b1, box_w1, box_b1,
           cls_w2, cls_b2, box_w2, box_b2,
           cls_w3, cls_b3, box_w3, box_b3,
           score_w, score_b, pred_w, pred_b):
    bf = jnp.bfloat16
    feats = [feat_p3[0], feat_p4[0], feat_p5[0], feat_p6[0], feat_p7[0]]
    segs = []
    cur = 0
    for (h, w), pw, s, f in zip(_LEVELS, _PWS, _STARTS, feats):
        segs.append(jnp.zeros((s - cur, C), bf))
        xp = jnp.pad(f, ((1, 1), (1, 1), (0, 0)))
        segs.append(xp.reshape((h + 2) * pw, C).astype(bf))
        cur = s + (h + 2) * pw
    segs.append(jnp.zeros((_ROWS - cur, C), bf))
    xin = jnp.concatenate(segs, axis=0)

    clsw = jnp.stack([cls_w0, cls_w1, cls_w2, cls_w3]).reshape(36, C, C)
    boxw = jnp.stack([box_w0, box_w1, box_w2, box_w3]).reshape(36, C, C)
    tww = jnp.stack([clsw, boxw]).astype(bf)
    towb = jnp.stack([jnp.stack([cls_b0, cls_b1, cls_b2, cls_b3]),
                      jnp.stack([box_b0, box_b1, box_b2, box_b3])])
    scw = jnp.pad(score_w, ((0, 0), (0, 0), (0, 0), (0, 48))).reshape(9, C, 768)
    prw = jnp.pad(pred_w, ((0, 0), (0, 0), (0, 0), (0, 732))).reshape(9, C, 768)
    headw = jnp.stack([scw, prw]).astype(bf)
    headb = jnp.stack([jnp.pad(score_b, (0, 48)).reshape(1, 768),
                       jnp.pad(pred_b, (0, 732)).reshape(1, 768)])

    res = pl.pallas_call(
        _head_kern,
        grid=(2,),
        in_specs=[
            pl.BlockSpec((_ROWS, C), lambda i: (0, 0)),
            pl.BlockSpec((1, 36, C, C), lambda i: (i, 0, 0, 0)),
            pl.BlockSpec((1, 4, C), lambda i: (i, 0, 0)),
            pl.BlockSpec((1, 9, C, 768), lambda i: (i, 0, 0, 0)),
            pl.BlockSpec((1, 1, 768), lambda i: (i, 0, 0)),
        ],
        out_specs=pl.BlockSpec((1, _ROWS, 768), lambda i: (i, 0, 0)),
        out_shape=jax.ShapeDtypeStruct((2, _ROWS, 768), jnp.float32),
        scratch_shapes=[pltpu.VMEM((_ROWS, C), bf),
                        pltpu.VMEM((_ROWS, C), bf)],
        compiler_params=pltpu.CompilerParams(
            dimension_semantics=("parallel",),
            vmem_limit_bytes=63 * 1024 * 1024),
    )(xin, tww, towb, headw, headb)

    cls_parts, box_parts = [], []
    for (h, w), pw, s in zip(_LEVELS, _PWS, _STARTS):
        n = (h + 2) * pw
        c3 = res[0, s:s + n].reshape(h + 2, pw, 768)[1:h + 1, 1:w + 1, :720]
        cls_parts.append(c3.reshape(h * w * 9, 80))
        b3 = res[1, s:s + n].reshape(h + 2, pw, 768)[1:h + 1, 1:w + 1, :36]
        box_parts.append(b3.reshape(h * w * 9, 4))
    return jnp.concatenate(cls_parts, 0), jnp.concatenate(box_parts, 0)


# speedup vs baseline: 1.3148x; 1.3148x over previous
import jax
import jax.numpy as jnp
from jax.experimental import pallas as pl
from jax.experimental.pallas import tpu as pltpu

C = 256
_LEVELS = [(48, 48), (24, 24), (12, 12), (6, 6), (3, 3)]
_PWS = [w + 2 for _, w in _LEVELS]
_CH = 2048

_STARTS = []
_cur = 5
for (_h, _w), _pw in zip(_LEVELS, _PWS):
    _s = _cur + (-( _cur + _pw + 1)) % 8
    _STARTS.append(_s)
    _cur = _s + (_h + 2) * _pw
_ROWS = _cur + (-_cur) % 8 + 8

_SPANS = []
_edges = []
for _l, ((_h, _w), _pw, _s) in enumerate(zip(_LEVELS, _PWS, _STARTS)):
    _a = _s + _pw + 1
    _L = (_h - 1) * _pw + _w
    _L += (-_L) % 8
    _edges.append((_a, _a + _L))
    for _c0 in range(_a, _a + _L, _CH):
        _SPANS.append((_l, _c0, min(_c0 + _CH, _a + _L)))
_HOLES = []
_prev = 0
for _a, _b in _edges:
    if _a > _prev:
        _HOLES.append((_prev, _a))
    _prev = _b
_HOLES.append((_prev, _ROWS))


def _mask_chunk(lvl, c0, c1):
    (h, w), pw, s = _LEVELS[lvl], _PWS[lvl], _STARTS[lvl]
    r = jax.lax.broadcasted_iota(jnp.int32, (c1 - c0, 1), 0) + c0
    ww = (r - s) % pw
    return (ww >= 1) & (ww <= w) & (r >= s + pw) & (r < s + (h + 1) * pw)


def _head_kern(xin, clsw, clsb, boxw, boxb, scw, scb, prw, prb,
               oc, ob, sa, sbuf):
    for buf in (sa, sbuf):
        for a, b in _HOLES:
            buf[a:b, :] = jnp.zeros((b - a, C), jnp.bfloat16)

    def conv(src, w_ref, base, b_row, dst, relu):
        for lvl, c0, c1 in _SPANS:
            pw = _PWS[lvl]
            acc = None
            for t, (dy, dx) in enumerate(
                    (dy, dx) for dy in (-1, 0, 1) for dx in (-1, 0, 1)):
                off = dy * pw + dx
                d = jnp.dot(src[c0 + off:c1 + off, :], w_ref[base + t],
                            preferred_element_type=jnp.float32)
                acc = d if acc is None else acc + d
            y = acc + b_row
            if relu:
                y = jnp.where(_mask_chunk(lvl, c0, c1),
                              jnp.maximum(y, 0.0), 0.0)
                dst[c0:c1, :] = y.astype(jnp.bfloat16)
            else:
                dst[c0:c1, :] = y

    def tower(tw, tb, hw, hb, out_ref):
        src = xin
        for i in range(4):
            dst = sa if i % 2 == 0 else sbuf
            conv(src, tw, 9 * i, tb[i:i + 1, :], dst, True)
            src = dst
        conv(src, hw, 0, hb[0:1, :], out_ref, False)

    tower(clsw, clsb, scw, scb, oc)
    tower(boxw, boxb, prw, prb, ob)


def kernel(feat_p3, feat_p4, feat_p5, feat_p6, feat_p7,
           cls_w0, cls_b0, box_w0, box_b0,
           cls_w1, cls_b1, box_w1, box_b1,
           cls_w2, cls_b2, box_w2, box_b2,
           cls_w3, cls_b3, box_w3, box_b3,
           score_w, score_b, pred_w, pred_b):
    bf = jnp.bfloat16
    feats = [feat_p3[0], feat_p4[0], feat_p5[0], feat_p6[0], feat_p7[0]]
    segs = []
    cur = 0
    for (h, w), pw, s, f in zip(_LEVELS, _PWS, _STARTS, feats):
        segs.append(jnp.zeros((s - cur, C), bf))
        xp = jnp.pad(f, ((1, 1), (1, 1), (0, 0)))
        segs.append(xp.reshape((h + 2) * pw, C).astype(bf))
        cur = s + (h + 2) * pw
    segs.append(jnp.zeros((_ROWS - cur, C), bf))
    xin = jnp.concatenate(segs, axis=0)

    clsw = jnp.stack([cls_w0, cls_w1, cls_w2, cls_w3]).reshape(36, C, C).astype(bf)
    clsb = jnp.stack([cls_b0, cls_b1, cls_b2, cls_b3])
    boxw = jnp.stack([box_w0, box_w1, box_w2, box_w3]).reshape(36, C, C).astype(bf)
    boxb = jnp.stack([box_b0, box_b1, box_b2, box_b3])
    scw = jnp.pad(score_w, ((0, 0), (0, 0), (0, 0), (0, 48))).reshape(9, C, 768).astype(bf)
    scb = jnp.pad(score_b, (0, 48)).reshape(1, 768)
    prw = jnp.pad(pred_w, ((0, 0), (0, 0), (0, 0), (0, 92))).reshape(9, C, 128).astype(bf)
    prb = jnp.pad(pred_b, (0, 92)).reshape(1, 128)

    oc, ob = pl.pallas_call(
        _head_kern,
        out_shape=[jax.ShapeDtypeStruct((_ROWS, 768), jnp.float32),
                   jax.ShapeDtypeStruct((_ROWS, 128), jnp.float32)],
        scratch_shapes=[pltpu.VMEM((_ROWS, C), bf),
                        pltpu.VMEM((_ROWS, C), bf)],
        compiler_params=pltpu.CompilerParams(
            vmem_limit_bytes=63 * 1024 * 1024),
    )(xin, clsw, clsb, boxw, boxb, scw, scb, prw, prb)

    cls_parts, box_parts = [], []
    for (h, w), pw, s in zip(_LEVELS, _PWS, _STARTS):
        n = (h + 2) * pw
        c3 = oc[s:s + n].reshape(h + 2, pw, 768)[1:h + 1, 1:w + 1, :720]
        cls_parts.append(c3.reshape(h * w * 9, 80))
        b3 = ob[s:s + n].reshape(h + 2, pw, 128)[1:h + 1, 1:w + 1, :36]
        box_parts.append(b3.reshape(h * w * 9, 4))
    return jnp.concatenate(cls_parts, 0), jnp.concatenate(box_parts, 0)


# speedup vs baseline: 1.3387x; 1.0182x over previous
import jax
import jax.numpy as jnp
from jax.experimental import pallas as pl
from jax.experimental.pallas import tpu as pltpu

C = 256
_LEVELS = [(48, 48), (24, 24), (12, 12), (6, 6), (3, 3)]
_PWS = [w + 2 for _, w in _LEVELS]
_CH = 512

_STARTS = []
_cur = 5
for (_h, _w), _pw in zip(_LEVELS, _PWS):
    _s = _cur + (-( _cur + _pw + 1)) % 8
    _STARTS.append(_s)
    _cur = _s + (_h + 2) * _pw
_ROWS = _cur + (-_cur) % 8 + 8

_SPANS = []
_edges = []
for _l, ((_h, _w), _pw, _s) in enumerate(zip(_LEVELS, _PWS, _STARTS)):
    _a = _s + _pw + 1
    _L = (_h - 1) * _pw + _w
    _L += (-_L) % 8
    _edges.append((_a, _a + _L))
    for _c0 in range(_a, _a + _L, _CH):
        _SPANS.append((_l, _c0, min(_c0 + _CH, _a + _L)))
_HOLES = []
_prev = 0
for _a, _b in _edges:
    if _a > _prev:
        _HOLES.append((_prev, _a))
    _prev = _b
_HOLES.append((_prev, _ROWS))


def _mask_chunk(lvl, c0, c1):
    (h, w), pw, s = _LEVELS[lvl], _PWS[lvl], _STARTS[lvl]
    r = jax.lax.broadcasted_iota(jnp.int32, (c1 - c0, 1), 0) + c0
    ww = (r - s) % pw
    return (ww >= 1) & (ww <= w) & (r >= s + pw) & (r < s + (h + 1) * pw)


def _head_kern(xin, clsw, clsb, boxw, boxb, scw, scb, prw, prb,
               oc, ob, sa, sbuf):
    for buf in (sa, sbuf):
        for a, b in _HOLES:
            buf[a:b, :] = jnp.zeros((b - a, C), jnp.bfloat16)

    def conv(src, w_ref, base, b_row, dst, relu):
        for lvl, c0, c1 in _SPANS:
            pw = _PWS[lvl]
            acc = None
            for t, (dy, dx) in enumerate(
                    (dy, dx) for dy in (-1, 0, 1) for dx in (-1, 0, 1)):
                off = dy * pw + dx
                d = jnp.dot(src[c0 + off:c1 + off, :], w_ref[base + t],
                            preferred_element_type=jnp.float32)
                acc = d if acc is None else acc + d
            y = acc + b_row
            if relu:
                y = jnp.where(_mask_chunk(lvl, c0, c1),
                              jnp.maximum(y, 0.0), 0.0)
                dst[c0:c1, :] = y.astype(jnp.bfloat16)
            else:
                dst[c0:c1, :] = y

    def tower(tw, tb, hw, hb, out_ref):
        src = xin
        for i in range(4):
            dst = sa if i % 2 == 0 else sbuf
            conv(src, tw, 9 * i, tb[i:i + 1, :], dst, True)
            src = dst
        conv(src, hw, 0, hb[0:1, :], out_ref, False)

    tower(clsw, clsb, scw, scb, oc)
    tower(boxw, boxb, prw, prb, ob)


def kernel(feat_p3, feat_p4, feat_p5, feat_p6, feat_p7,
           cls_w0, cls_b0, box_w0, box_b0,
           cls_w1, cls_b1, box_w1, box_b1,
           cls_w2, cls_b2, box_w2, box_b2,
           cls_w3, cls_b3, box_w3, box_b3,
           score_w, score_b, pred_w, pred_b):
    bf = jnp.bfloat16
    feats = [feat_p3[0], feat_p4[0], feat_p5[0], feat_p6[0], feat_p7[0]]
    segs = []
    cur = 0
    for (h, w), pw, s, f in zip(_LEVELS, _PWS, _STARTS, feats):
        segs.append(jnp.zeros((s - cur, C), bf))
        xp = jnp.pad(f, ((1, 1), (1, 1), (0, 0)))
        segs.append(xp.reshape((h + 2) * pw, C).astype(bf))
        cur = s + (h + 2) * pw
    segs.append(jnp.zeros((_ROWS - cur, C), bf))
    xin = jnp.concatenate(segs, axis=0)

    clsw = jnp.stack([cls_w0, cls_w1, cls_w2, cls_w3]).reshape(36, C, C).astype(bf)
    clsb = jnp.stack([cls_b0, cls_b1, cls_b2, cls_b3])
    boxw = jnp.stack([box_w0, box_w1, box_w2, box_w3]).reshape(36, C, C).astype(bf)
    boxb = jnp.stack([box_b0, box_b1, box_b2, box_b3])
    scw = jnp.pad(score_w, ((0, 0), (0, 0), (0, 0), (0, 48))).reshape(9, C, 768).astype(bf)
    scb = jnp.pad(score_b, (0, 48)).reshape(1, 768)
    prw = jnp.pad(pred_w, ((0, 0), (0, 0), (0, 0), (0, 92))).reshape(9, C, 128).astype(bf)
    prb = jnp.pad(pred_b, (0, 92)).reshape(1, 128)

    oc, ob = pl.pallas_call(
        _head_kern,
        out_shape=[jax.ShapeDtypeStruct((_ROWS, 768), jnp.float32),
                   jax.ShapeDtypeStruct((_ROWS, 128), jnp.float32)],
        scratch_shapes=[pltpu.VMEM((_ROWS, C), bf),
                        pltpu.VMEM((_ROWS, C), bf)],
        compiler_params=pltpu.CompilerParams(
            vmem_limit_bytes=63 * 1024 * 1024),
    )(xin, clsw, clsb, boxw, boxb, scw, scb, prw, prb)

    cls_parts, box_parts = [], []
    for (h, w), pw, s in zip(_LEVELS, _PWS, _STARTS):
        n = (h + 2) * pw
        c3 = oc[s:s + n].reshape(h + 2, pw, 768)[1:h + 1, 1:w + 1, :720]
        cls_parts.append(c3.reshape(h * w * 9, 80))
        b3 = ob[s:s + n].reshape(h + 2, pw, 128)[1:h + 1, 1:w + 1, :36]
        box_parts.append(b3.reshape(h * w * 9, 4))
    return jnp.concatenate(cls_parts, 0), jnp.concatenate(box_parts, 0)


# speedup vs baseline: 1.3484x; 1.0072x over previous
import jax
import jax.numpy as jnp
from jax.experimental import pallas as pl
from jax.experimental.pallas import tpu as pltpu

C = 256
_LEVELS = [(48, 48), (24, 24), (12, 12), (6, 6), (3, 3)]
_PWS = [56, 26, 14, 8, 5]
_CH = 1024

_STARTS = []
_cur = 5
for (_h, _w), _pw in zip(_LEVELS, _PWS):
    _s = _cur + (-( _cur + _pw + 1)) % 8
    _STARTS.append(_s)
    _cur = _s + (_h + 2) * _pw
_ROWS = _cur + (-_cur) % 8 + 8

_SPANS = []
_edges = []
for _l, ((_h, _w), _pw, _s) in enumerate(zip(_LEVELS, _PWS, _STARTS)):
    _a = _s + _pw + 1
    _L = (_h - 1) * _pw + _w
    _L += (-_L) % 8
    _edges.append((_a, _a + _L))
    for _c0 in range(_a, _a + _L, _CH):
        _SPANS.append((_l, _c0, min(_c0 + _CH, _a + _L)))
_HOLES = []
_prev = 0
for _a, _b in _edges:
    if _a > _prev:
        _HOLES.append((_prev, _a))
    _prev = _b
_HOLES.append((_prev, _ROWS))


def _mask_chunk(lvl, c0, c1):
    (h, w), pw, s = _LEVELS[lvl], _PWS[lvl], _STARTS[lvl]
    r = jax.lax.broadcasted_iota(jnp.int32, (c1 - c0, 1), 0) + c0
    ww = (r - s) % pw
    return (ww >= 1) & (ww <= w) & (r >= s + pw) & (r < s + (h + 1) * pw)


def _head_kern(xin, clsw, clsb, boxw, boxb, scw, scb, prw, prb,
               oc, ob, sa, sbuf):
    for buf in (sa, sbuf):
        for a, b in _HOLES:
            buf[a:b, :] = jnp.zeros((b - a, C), jnp.bfloat16)

    def conv(src, w_ref, base, b_row, dst, relu):
        for lvl, c0, c1 in _SPANS:
            pw = _PWS[lvl]
            acc = None
            for t, (dy, dx) in enumerate(
                    (dy, dx) for dy in (-1, 0, 1) for dx in (-1, 0, 1)):
                off = dy * pw + dx
                d = jnp.dot(src[c0 + off:c1 + off, :], w_ref[base + t],
                            preferred_element_type=jnp.float32)
                acc = d if acc is None else acc + d
            y = acc + b_row
            if relu:
                y = jnp.where(_mask_chunk(lvl, c0, c1),
                              jnp.maximum(y, 0.0), 0.0)
                dst[c0:c1, :] = y.astype(jnp.bfloat16)
            else:
                dst[c0:c1, :] = y

    def tower(tw, tb, hw, hb, out_ref):
        src = xin
        for i in range(4):
            dst = sa if i % 2 == 0 else sbuf
            conv(src, tw, 9 * i, tb[i:i + 1, :], dst, True)
            src = dst
        conv(src, hw, 0, hb[0:1, :], out_ref, False)

    tower(clsw, clsb, scw, scb, oc)
    tower(boxw, boxb, prw, prb, ob)


def kernel(feat_p3, feat_p4, feat_p5, feat_p6, feat_p7,
           cls_w0, cls_b0, box_w0, box_b0,
           cls_w1, cls_b1, box_w1, box_b1,
           cls_w2, cls_b2, box_w2, box_b2,
           cls_w3, cls_b3, box_w3, box_b3,
           score_w, score_b, pred_w, pred_b):
    bf = jnp.bfloat16
    feats = [feat_p3[0], feat_p4[0], feat_p5[0], feat_p6[0], feat_p7[0]]
    segs = []
    cur = 0
    for (h, w), pw, s, f in zip(_LEVELS, _PWS, _STARTS, feats):
        segs.append(jnp.zeros((s - cur, C), bf))
        xp = jnp.pad(f, ((1, 1), (1, pw - w - 1), (0, 0)))
        segs.append(xp.reshape((h + 2) * pw, C).astype(bf))
        cur = s + (h + 2) * pw
    segs.append(jnp.zeros((_ROWS - cur, C), bf))
    xin = jnp.concatenate(segs, axis=0)

    clsw = jnp.stack([cls_w0, cls_w1, cls_w2, cls_w3]).reshape(36, C, C).astype(bf)
    clsb = jnp.stack([cls_b0, cls_b1, cls_b2, cls_b3])
    boxw = jnp.stack([box_w0, box_w1, box_w2, box_w3]).reshape(36, C, C).astype(bf)
    boxb = jnp.stack([box_b0, box_b1, box_b2, box_b3])
    scw = jnp.pad(score_w, ((0, 0), (0, 0), (0, 0), (0, 48))).reshape(9, C, 768).astype(bf)
    scb = jnp.pad(score_b, (0, 48)).reshape(1, 768)
    prw = jnp.pad(pred_w, ((0, 0), (0, 0), (0, 0), (0, 92))).reshape(9, C, 128).astype(bf)
    prb = jnp.pad(pred_b, (0, 92)).reshape(1, 128)

    oc, ob = pl.pallas_call(
        _head_kern,
        out_shape=[jax.ShapeDtypeStruct((_ROWS, 768), jnp.float32),
                   jax.ShapeDtypeStruct((_ROWS, 128), jnp.float32)],
        scratch_shapes=[pltpu.VMEM((_ROWS, C), bf),
                        pltpu.VMEM((_ROWS, C), bf)],
        compiler_params=pltpu.CompilerParams(
            vmem_limit_bytes=63 * 1024 * 1024),
    )(xin, clsw, clsb, boxw, boxb, scw, scb, prw, prb)

    cls_parts, box_parts = [], []
    for (h, w), pw, s in zip(_LEVELS, _PWS, _STARTS):
        n = (h + 2) * pw
        c3 = oc[s:s + n].reshape(h + 2, pw, 768)[1:h + 1, 1:w + 1, :720]
        cls_parts.append(c3.reshape(h * w * 9, 80))
        b3 = ob[s:s + n].reshape(h + 2, pw, 128)[1:h + 1, 1:w + 1, :36]
        box_parts.append(b3.reshape(h * w * 9, 4))
    return jnp.concatenate(cls_parts, 0), jnp.concatenate(box_parts, 0)


# speedup vs baseline: 1.3715x; 1.0171x over previous
import jax
import jax.numpy as jnp
from jax.experimental import pallas as pl
from jax.experimental.pallas import tpu as pltpu

C = 256
_LEVELS = [(48, 48), (24, 24), (12, 12), (6, 6), (3, 3)]
_PWS = [56, 32, 16, 8, 8]
_CH = 1024

_STARTS = []
_cur = 5
for (_h, _w), _pw in zip(_LEVELS, _PWS):
    _s = _cur + (-( _cur + _pw + 1)) % 8
    _STARTS.append(_s)
    _cur = _s + (_h + 2) * _pw
_ROWS = _cur + (-_cur) % 8 + 8

_SPANS = []
_edges = []
for _l, ((_h, _w), _pw, _s) in enumerate(zip(_LEVELS, _PWS, _STARTS)):
    _a = _s + _pw + 1
    _L = (_h - 1) * _pw + _w
    _L += (-_L) % 8
    _edges.append((_a, _a + _L))
    for _c0 in range(_a, _a + _L, _CH):
        _SPANS.append((_l, _c0, min(_c0 + _CH, _a + _L)))
_HOLES = []
_prev = 0
for _a, _b in _edges:
    if _a > _prev:
        _HOLES.append((_prev, _a))
    _prev = _b
_HOLES.append((_prev, _ROWS))


def _mask_chunk(lvl, c0, c1):
    (h, w), pw, s = _LEVELS[lvl], _PWS[lvl], _STARTS[lvl]
    r = jax.lax.broadcasted_iota(jnp.int32, (c1 - c0, 1), 0) + c0
    ww = (r - s) % pw
    return (ww >= 1) & (ww <= w) & (r >= s + pw) & (r < s + (h + 1) * pw)


def _head_kern(xin, clsw, clsb, boxw, boxb, scw, scb, prw, prb,
               oc, ob, sa, sbuf):
    for buf in (sa, sbuf):
        for a, b in _HOLES:
            buf[a:b, :] = jnp.zeros((b - a, C), jnp.bfloat16)

    def conv(src, w_ref, base, b_row, dst, relu):
        for lvl, c0, c1 in _SPANS:
            pw = _PWS[lvl]
            acc = None
            for t, (dy, dx) in enumerate(
                    (dy, dx) for dy in (-1, 0, 1) for dx in (-1, 0, 1)):
                off = dy * pw + dx
                d = jnp.dot(src[c0 + off:c1 + off, :], w_ref[base + t],
                            preferred_element_type=jnp.float32)
                acc = d if acc is None else acc + d
            y = acc + b_row
            if relu:
                y = jnp.where(_mask_chunk(lvl, c0, c1),
                              jnp.maximum(y, 0.0), 0.0)
                dst[c0:c1, :] = y.astype(jnp.bfloat16)
            else:
                dst[c0:c1, :] = y

    def tower(tw, tb, hw, hb, out_ref):
        src = xin
        for i in range(4):
            dst = sa if i % 2 == 0 else sbuf
            conv(src, tw, 9 * i, tb[i:i + 1, :], dst, True)
            src = dst
        conv(src, hw, 0, hb[0:1, :], out_ref, False)

    tower(clsw, clsb, scw, scb, oc)
    tower(boxw, boxb, prw, prb, ob)


def kernel(feat_p3, feat_p4, feat_p5, feat_p6, feat_p7,
           cls_w0, cls_b0, box_w0, box_b0,
           cls_w1, cls_b1, box_w1, box_b1,
           cls_w2, cls_b2, box_w2, box_b2,
           cls_w3, cls_b3, box_w3, box_b3,
           score_w, score_b, pred_w, pred_b):
    bf = jnp.bfloat16
    feats = [feat_p3[0], feat_p4[0], feat_p5[0], feat_p6[0], feat_p7[0]]
    segs = []
    cur = 0
    for (h, w), pw, s, f in zip(_LEVELS, _PWS, _STARTS, feats):
        segs.append(jnp.zeros((s - cur, C), bf))
        xp = jnp.pad(f, ((1, 1), (1, pw - w - 1), (0, 0)))
        segs.append(xp.reshape((h + 2) * pw, C).astype(bf))
        cur = s + (h + 2) * pw
    segs.append(jnp.zeros((_ROWS - cur, C), bf))
    xin = jnp.concatenate(segs, axis=0)

    clsw = jnp.stack([cls_w0, cls_w1, cls_w2, cls_w3]).reshape(36, C, C).astype(bf)
    clsb = jnp.stack([cls_b0, cls_b1, cls_b2, cls_b3])
    boxw = jnp.stack([box_w0, box_w1, box_w2, box_w3]).reshape(36, C, C).astype(bf)
    boxb = jnp.stack([box_b0, box_b1, box_b2, box_b3])
    scw = jnp.pad(score_w, ((0, 0), (0, 0), (0, 0), (0, 48))).reshape(9, C, 768).astype(bf)
    scb = jnp.pad(score_b, (0, 48)).reshape(1, 768)
    prw = jnp.pad(pred_w, ((0, 0), (0, 0), (0, 0), (0, 92))).reshape(9, C, 128).astype(bf)
    prb = jnp.pad(pred_b, (0, 92)).reshape(1, 128)

    oc, ob = pl.pallas_call(
        _head_kern,
        out_shape=[jax.ShapeDtypeStruct((_ROWS, 768), jnp.float32),
                   jax.ShapeDtypeStruct((_ROWS, 128), jnp.float32)],
        scratch_shapes=[pltpu.VMEM((_ROWS, C), bf),
                        pltpu.VMEM((_ROWS, C), bf)],
        compiler_params=pltpu.CompilerParams(
            vmem_limit_bytes=63 * 1024 * 1024),
    )(xin, clsw, clsb, boxw, boxb, scw, scb, prw, prb)

    cls_parts, box_parts = [], []
    for (h, w), pw, s in zip(_LEVELS, _PWS, _STARTS):
        n = (h + 2) * pw
        c3 = oc[s:s + n].reshape(h + 2, pw, 768)[1:h + 1, 1:w + 1, :720]
        cls_parts.append(c3.reshape(h * w * 9, 80))
        b3 = ob[s:s + n].reshape(h + 2, pw, 128)[1:h + 1, 1:w + 1, :36]
        box_parts.append(b3.reshape(h * w * 9, 4))
    return jnp.concatenate(cls_parts, 0), jnp.concatenate(box_parts, 0)


# speedup vs baseline: 1.3908x; 1.0141x over previous
import jax
import jax.numpy as jnp
from jax.experimental import pallas as pl
from jax.experimental.pallas import tpu as pltpu

C = 256
_LEVELS = [(48, 48), (24, 24), (12, 12), (6, 6), (3, 3)]
_PWS = [56, 32, 16, 8, 8]
_CH = 1024

_STARTS = []
_cur = 5
for (_h, _w), _pw in zip(_LEVELS, _PWS):
    _s = _cur + (-( _cur + _pw + 1)) % 8
    _STARTS.append(_s)
    _cur = _s + (_h + 2) * _pw
_ROWS = _cur + (-_cur) % 8 + 8

_SPANS = []
_edges = []
for _l, ((_h, _w), _pw, _s) in enumerate(zip(_LEVELS, _PWS, _STARTS)):
    _a = _s + _pw + 1
    _L = (_h - 1) * _pw + _w
    _L += (-_L) % 8
    _edges.append((_a, _a + _L))
    for _c0 in range(_a, _a + _L, _CH):
        _SPANS.append((_l, _c0, min(_c0 + _CH, _a + _L)))
_HOLES = []
_prev = 0
for _a, _b in _edges:
    if _a > _prev:
        _HOLES.append((_prev, _a))
    _prev = _b
_HOLES.append((_prev, _ROWS))
_HOLES_P1 = [(max(p - 1, 0), q - 1) for p, q in _HOLES[:-1]] + \
    [(_HOLES[-1][0] - 1, _ROWS)]
_HOLES_M1 = [(0, _HOLES[0][1] + 1)] + \
    [(p + 1, min(q + 1, _ROWS)) for p, q in _HOLES[1:]]


def _mask_chunk(lvl, c0, c1):
    (h, w), pw, s = _LEVELS[lvl], _PWS[lvl], _STARTS[lvl]
    r = jax.lax.broadcasted_iota(jnp.int32, (c1 - c0, 1), 0) + c0
    ww = (r - s) % pw
    return (ww >= 1) & (ww <= w) & (r >= s + pw) & (r < s + (h + 1) * pw)


def _head_kern(xin, xinp, xinm, clsw, clsb, boxw, boxb, scw, scb, prw, prb,
               oc, ob, sa, sap, sam, sbuf, sbp, sbm):
    for buf, holes in ((sa, _HOLES), (sbuf, _HOLES),
                       (sap, _HOLES_P1), (sbp, _HOLES_P1),
                       (sam, _HOLES_M1), (sbm, _HOLES_M1)):
        for a, b in holes:
            buf[a:b, :] = jnp.zeros((b - a, C), jnp.bfloat16)

    def conv(srcs, w_ref, base, b_row, dsts, relu):
        s0, sp1, sm1 = srcs
        for lvl, c0, c1 in _SPANS:
            pw = _PWS[lvl]
            acc = None
            for t, (dy, dx) in enumerate(
                    (dy, dx) for dy in (-1, 0, 1) for dx in (-1, 0, 1)):
                src = s0 if dx == 0 else (sp1 if dx == 1 else sm1)
                off = dy * pw
                d = jnp.dot(src[c0 + off:c1 + off, :], w_ref[base + t],
                            preferred_element_type=jnp.float32)
                acc = d if acc is None else acc + d
            y = acc + b_row
            if relu:
                y = jnp.where(_mask_chunk(lvl, c0, c1),
                              jnp.maximum(y, 0.0), 0.0)
                yb = y.astype(jnp.bfloat16)
                d0, dp1, dm1 = dsts
                d0[c0:c1, :] = yb
                dp1[c0 - 1:c1 - 1, :] = yb
                dm1[c0 + 1:c1 + 1, :] = yb
            else:
                dsts[c0:c1, :] = y

    def tower(tw, tb, hw, hb, out_ref):
        srcs = (xin, xinp, xinm)
        for i in range(4):
            dsts = (sa, sap, sam) if i % 2 == 0 else (sbuf, sbp, sbm)
            conv(srcs, tw, 9 * i, tb[i:i + 1, :], dsts, True)
            srcs = dsts
        conv(srcs, hw, 0, hb[0:1, :], out_ref, False)

    tower(clsw, clsb, scw, scb, oc)
    tower(boxw, boxb, prw, prb, ob)


def kernel(feat_p3, feat_p4, feat_p5, feat_p6, feat_p7,
           cls_w0, cls_b0, box_w0, box_b0,
           cls_w1, cls_b1, box_w1, box_b1,
           cls_w2, cls_b2, box_w2, box_b2,
           cls_w3, cls_b3, box_w3, box_b3,
           score_w, score_b, pred_w, pred_b):
    bf = jnp.bfloat16
    feats = [feat_p3[0], feat_p4[0], feat_p5[0], feat_p6[0], feat_p7[0]]
    segs = []
    cur = 0
    for (h, w), pw, s, f in zip(_LEVELS, _PWS, _STARTS, feats):
        segs.append(jnp.zeros((s - cur, C), bf))
        xp = jnp.pad(f, ((1, 1), (1, pw - w - 1), (0, 0)))
        segs.append(xp.reshape((h + 2) * pw, C).astype(bf))
        cur = s + (h + 2) * pw
    segs.append(jnp.zeros((_ROWS - cur, C), bf))
    xin = jnp.concatenate(segs, axis=0)
    z1 = jnp.zeros((1, C), bf)
    xinp = jnp.concatenate([xin[1:], z1], axis=0)
    xinm = jnp.concatenate([z1, xin[:-1]], axis=0)

    clsw = jnp.stack([cls_w0, cls_w1, cls_w2, cls_w3]).reshape(36, C, C).astype(bf)
    clsb = jnp.stack([cls_b0, cls_b1, cls_b2, cls_b3])
    boxw = jnp.stack([box_w0, box_w1, box_w2, box_w3]).reshape(36, C, C).astype(bf)
    boxb = jnp.stack([box_b0, box_b1, box_b2, box_b3])
    scw = jnp.pad(score_w, ((0, 0), (0, 0), (0, 0), (0, 48))).reshape(9, C, 768).astype(bf)
    scb = jnp.pad(score_b, (0, 48)).reshape(1, 768)
    prw = jnp.pad(pred_w, ((0, 0), (0, 0), (0, 0), (0, 92))).reshape(9, C, 128).astype(bf)
    prb = jnp.pad(pred_b, (0, 92)).reshape(1, 128)

    oc, ob = pl.pallas_call(
        _head_kern,
        out_shape=[jax.ShapeDtypeStruct((_ROWS, 768), jnp.float32),
                   jax.ShapeDtypeStruct((_ROWS, 128), jnp.float32)],
        scratch_shapes=[pltpu.VMEM((_ROWS, C), bf)] * 6,
        compiler_params=pltpu.CompilerParams(
            vmem_limit_bytes=63 * 1024 * 1024),
    )(xin, xinp, xinm, clsw, clsb, boxw, boxb, scw, scb, prw, prb)

    cls_parts, box_parts = [], []
    for (h, w), pw, s in zip(_LEVELS, _PWS, _STARTS):
        n = (h + 2) * pw
        c3 = oc[s:s + n].reshape(h + 2, pw, 768)[1:h + 1, 1:w + 1, :720]
        cls_parts.append(c3.reshape(h * w * 9, 80))
        b3 = ob[s:s + n].reshape(h + 2, pw, 128)[1:h + 1, 1:w + 1, :36]
        box_parts.append(b3.reshape(h * w * 9, 4))
    return jnp.concatenate(cls_parts, 0), jnp.concatenate(box_parts, 0)


# speedup vs baseline: 1.5758x; 1.1330x over previous
import jax
import jax.numpy as jnp
from jax.experimental import pallas as pl
from jax.experimental.pallas import tpu as pltpu

C = 256
_LEVELS = [(48, 48), (24, 24), (12, 12), (6, 6), (3, 3)]
_PWS = [56, 32, 16, 8, 8]
_CH = 1024

_STARTS = []
_cur = 5
for (_h, _w), _pw in zip(_LEVELS, _PWS):
    _s = _cur + (-( _cur + _pw + 1)) % 8
    _STARTS.append(_s)
    _cur = _s + (_h + 2) * _pw
_ROWS = _cur + (-_cur) % 8 + 8

_SPANS = []
_edges = []
for _l, ((_h, _w), _pw, _s) in enumerate(zip(_LEVELS, _PWS, _STARTS)):
    _a = _s + _pw + 1
    _L = (_h - 1) * _pw + _w
    _L += (-_L) % 8
    _edges.append((_a, _a + _L))
    for _c0 in range(_a, _a + _L, _CH):
        _SPANS.append((_l, _c0, min(_c0 + _CH, _a + _L)))
_HOLES = []
_prev = 0
for _a, _b in _edges:
    if _a > _prev:
        _HOLES.append((_prev, _a))
    _prev = _b
_HOLES.append((_prev, _ROWS))
_HOLES_P1 = [(max(p - 1, 0), q - 1) for p, q in _HOLES[:-1]] + \
    [(_HOLES[-1][0] - 1, _ROWS)]
_HOLES_M1 = [(0, _HOLES[0][1] + 1)] + \
    [(p + 1, min(q + 1, _ROWS)) for p, q in _HOLES[1:]]


def _mask_chunk(lvl, c0, c1):
    (h, w), pw, s = _LEVELS[lvl], _PWS[lvl], _STARTS[lvl]
    r = jax.lax.broadcasted_iota(jnp.int32, (c1 - c0, 1), 0) + c0
    ww = (r - s) % pw
    return (ww >= 1) & (ww <= w) & (r >= s + pw) & (r < s + (h + 1) * pw)


def _head_kern(xin, xinp, xinm, clsw, clsb, boxw, boxb, scw, scb, prw, prb,
               oc, ob, sa, sap, sam, sbuf, sbp, sbm):
    for buf, holes in ((sa, _HOLES), (sbuf, _HOLES),
                       (sap, _HOLES_P1), (sbp, _HOLES_P1),
                       (sam, _HOLES_M1), (sbm, _HOLES_M1)):
        for a, b in holes:
            buf[a:b, :] = jnp.zeros((b - a, C), jnp.bfloat16)

    def conv(srcs, w_ref, base, b_row, dsts, relu):
        s0, sp1, sm1 = srcs
        for lvl, c0, c1 in _SPANS:
            pw = _PWS[lvl]
            acc = None
            for t, (dy, dx) in enumerate(
                    (dy, dx) for dy in (-1, 0, 1) for dx in (-1, 0, 1)):
                src = s0 if dx == 0 else (sp1 if dx == 1 else sm1)
                off = dy * pw
                d = jnp.dot(src[c0 + off:c1 + off, :], w_ref[base + t],
                            preferred_element_type=jnp.float32)
                acc = d if acc is None else acc + d
            y = acc + b_row
            if relu:
                y = jnp.where(_mask_chunk(lvl, c0, c1),
                              jnp.maximum(y, 0.0), 0.0)
                yb = y.astype(jnp.bfloat16)
                d0, dp1, dm1 = dsts
                d0[c0:c1, :] = yb
                dp1[c0 - 1:c1 - 1, :] = yb
                dm1[c0 + 1:c1 + 1, :] = yb
            else:
                dsts[c0:c1, :] = y.astype(jnp.bfloat16)

    def tower(tw, tb, hw, hb, out_ref):
        srcs = (xin, xinp, xinm)
        for i in range(4):
            dsts = (sa, sap, sam) if i % 2 == 0 else (sbuf, sbp, sbm)
            conv(srcs, tw, 9 * i, tb[i:i + 1, :], dsts, True)
            srcs = dsts
        conv(srcs, hw, 0, hb[0:1, :], out_ref, False)

    tower(clsw, clsb, scw, scb, oc)
    tower(boxw, boxb, prw, prb, ob)


def kernel(feat_p3, feat_p4, feat_p5, feat_p6, feat_p7,
           cls_w0, cls_b0, box_w0, box_b0,
           cls_w1, cls_b1, box_w1, box_b1,
           cls_w2, cls_b2, box_w2, box_b2,
           cls_w3, cls_b3, box_w3, box_b3,
           score_w, score_b, pred_w, pred_b):
    bf = jnp.bfloat16
    feats = [feat_p3[0], feat_p4[0], feat_p5[0], feat_p6[0], feat_p7[0]]
    segs = []
    cur = 0
    for (h, w), pw, s, f in zip(_LEVELS, _PWS, _STARTS, feats):
        segs.append(jnp.zeros((s - cur, C), bf))
        xp = jnp.pad(f, ((1, 1), (1, pw - w - 1), (0, 0)))
        segs.append(xp.reshape((h + 2) * pw, C).astype(bf))
        cur = s + (h + 2) * pw
    segs.append(jnp.zeros((_ROWS - cur, C), bf))
    xin = jnp.concatenate(segs, axis=0)
    z1 = jnp.zeros((1, C), bf)
    xinp = jnp.concatenate([xin[1:], z1], axis=0)
    xinm = jnp.concatenate([z1, xin[:-1]], axis=0)

    clsw = jnp.stack([cls_w0, cls_w1, cls_w2, cls_w3]).reshape(36, C, C).astype(bf)
    clsb = jnp.stack([cls_b0, cls_b1, cls_b2, cls_b3])
    boxw = jnp.stack([box_w0, box_w1, box_w2, box_w3]).reshape(36, C, C).astype(bf)
    boxb = jnp.stack([box_b0, box_b1, box_b2, box_b3])
    scw = jnp.pad(score_w, ((0, 0), (0, 0), (0, 0), (0, 48))).reshape(9, C, 768).astype(bf)
    scb = jnp.pad(score_b, (0, 48)).reshape(1, 768)
    prw = jnp.pad(pred_w, ((0, 0), (0, 0), (0, 0), (0, 92))).reshape(9, C, 128).astype(bf)
    prb = jnp.pad(pred_b, (0, 92)).reshape(1, 128)

    oc, ob = pl.pallas_call(
        _head_kern,
        out_shape=[jax.ShapeDtypeStruct((_ROWS, 768), bf),
                   jax.ShapeDtypeStruct((_ROWS, 128), bf)],
        scratch_shapes=[pltpu.VMEM((_ROWS, C), bf)] * 6,
        compiler_params=pltpu.CompilerParams(
            vmem_limit_bytes=63 * 1024 * 1024),
    )(xin, xinp, xinm, clsw, clsb, boxw, boxb, scw, scb, prw, prb)

    cls_parts, box_parts = [], []
    for (h, w), pw, s in zip(_LEVELS, _PWS, _STARTS):
        n = (h + 2) * pw
        c3 = oc[s:s + n].reshape(h + 2, pw, 768)[1:h + 1, 1:w + 1, :720]
        cls_parts.append(c3.reshape(h * w * 9, 80))
        b3 = ob[s:s + n].reshape(h + 2, pw, 128)[1:h + 1, 1:w + 1, :36]
        box_parts.append(b3.reshape(h * w * 9, 4))
    return (jnp.concatenate(cls_parts, 0).astype(jnp.float32),
            jnp.concatenate(box_parts, 0).astype(jnp.float32))
